# Initial kernel scaffold; baseline (speedup 1.0000x reference)
import jax
import jax.numpy as jnp
from jax.experimental import pallas as pl


def kernel(x, edge_index, edge_attr, params):
    raise NotImplementedError("write your pallas kernel here")


# speedup vs baseline: 2.8253x; 2.8253x over previous
import functools

import jax
import jax.numpy as jnp
import numpy as np
from jax import lax
from jax.experimental import pallas as pl
from jax.experimental.pallas import tpu as pltpu
from jax.experimental.pallas import tpu_sc as plsc

F32 = jnp.float32
_N = 10000
_E = 320000
_H = 128
_BN = np.float32(1.0 / np.sqrt(1.0 + 1e-5))

_NW = 32
_EPW = _E // _NW
_C = 80
_NCH = _EPW // _C
_NPAD = 10240
_NPW = _NPAD // 16
_ZR = 128

_vmesh = plsc.VectorSubcoreMesh(core_axis_name="c", subcore_axis_name="s")


def _dot(a, b):
    return jnp.dot(a, b, preferred_element_type=F32)


def _relu(x):
    return jnp.maximum(x, 0.0)


def _mlp3_body(x_ref, w1, b1, w2, b2, w3, b3, o_ref):
    a = _relu((_dot(x_ref[...], w1[...]) + b1[...]) * _BN)
    a = _relu((_dot(a, w2[...]) + b2[...]) * _BN)
    o_ref[...] = _dot(a, w3[...]) + b3[...]


def _mlp3(xarr, mlp, block_rows):
    (w1, b1), (w2, b2), (w3, b3) = mlp
    rows = xarr.shape[0]
    dout = w3.shape[1]
    ws = [w1, b1.reshape(1, -1), w2, b2.reshape(1, -1), w3, b3.reshape(1, -1)]
    full = lambda a: pl.BlockSpec(a.shape, lambda i: (0, 0))
    return pl.pallas_call(
        _mlp3_body,
        grid=(rows // block_rows,),
        in_specs=[
            pl.BlockSpec((block_rows, xarr.shape[1]), lambda i: (i, 0)),
        ] + [full(w) for w in ws],
        out_specs=pl.BlockSpec((block_rows, dout), lambda i: (i, 0)),
        out_shape=jax.ShapeDtypeStruct((rows, dout), F32),
    )(xarr, *ws)


def _edge_body(hr_ref, hc_ref, e_ref,
               w1a, w1b, w1c, b1, w2, b2, w3, b3,
               wu1e, wu1m, bu1, wu2, bu2, wu3, bu3,
               eupd_ref, msgs_ref):
    hr = hr_ref[...]
    hc = hc_ref[...]
    e = e_ref[...]

    t1 = _dot(hr, w1a[...]) + _dot(hc, w1b[...]) + _dot(e, w1c[...]) + b1[...]
    a = _relu(t1 * _BN)
    a = _relu((_dot(a, w2[...]) + b2[...]) * _BN)
    em = _dot(a, w3[...]) + b3[...]

    u = _relu((_dot(e, wu1e[...]) + _dot(em, wu1m[...]) + bu1[...]) * _BN)
    u = _relu((_dot(u, wu2[...]) + bu2[...]) * _BN)
    e_upd = _dot(u, wu3[...]) + bu3[...]
    eupd_ref[...] = e_upd

    t2 = _dot(hc, w1a[...]) + _dot(hr, w1b[...]) + _dot(e_upd, w1c[...]) + b1[...]
    m = _relu(t2 * _BN)
    m = _relu((_dot(m, w2[...]) + b2[...]) * _BN)
    msgs_ref[...] = _dot(m, w3[...]) + b3[...]


def _tc_edge(hr, hc, e, edge_mlp, edge_update_mlp, block_rows=4000):
    (w1, b1), (w2, b2), (w3, b3) = edge_mlp
    (wu1, bu1), (wu2, bu2), (wu3, bu3) = edge_update_mlp
    w1a, w1b, w1c = w1[:_H], w1[_H:2 * _H], w1[2 * _H:]
    wu1e, wu1m = wu1[:_H // 2], wu1[_H // 2:]
    full = lambda a: pl.BlockSpec(a.shape, lambda i: (0, 0))
    ws = [w1a, w1b, w1c, b1.reshape(1, -1), w2, b2.reshape(1, -1),
          w3, b3.reshape(1, -1),
          wu1e, wu1m, bu1.reshape(1, -1), wu2, bu2.reshape(1, -1),
          wu3, bu3.reshape(1, -1)]
    return pl.pallas_call(
        _edge_body,
        grid=(_E // block_rows,),
        in_specs=[
            pl.BlockSpec((block_rows, _H), lambda i: (i, 0)),
            pl.BlockSpec((block_rows, _H), lambda i: (i, 0)),
            pl.BlockSpec((block_rows, _H // 2), lambda i: (i, 0)),
        ] + [full(w) for w in ws],
        out_specs=[
            pl.BlockSpec((block_rows, _H // 2), lambda i: (i, 0)),
            pl.BlockSpec((block_rows, _H), lambda i: (i, 0)),
        ],
        out_shape=[
            jax.ShapeDtypeStruct((_E, _H // 2), F32),
            jax.ShapeDtypeStruct((_E, _H), F32),
        ],
    )(hr, hc, e, *ws)


def _node_body(res, h_ref, p0_ref, p1_ref,
               wn1h, wn1a, bn1, wn2, bn2, wn3, bn3, o_ref):
    h = h_ref[...]
    agg = p0_ref[...] + p1_ref[...]
    a = _relu((_dot(h, wn1h[...]) + _dot(agg, wn1a[...]) + bn1[...]) * _BN)
    a = _relu((_dot(a, wn2[...]) + bn2[...]) * _BN)
    out = _dot(a, wn3[...]) + bn3[...]
    if res:
        out = out + h
    o_ref[...] = out


def _tc_node(h, p0, p1, node_mlp, residual, block_rows=2000):
    (wn1, bn1), (wn2, bn2), (wn3, bn3) = node_mlp
    wn1h, wn1a = wn1[:_H], wn1[_H:]
    full = lambda a: pl.BlockSpec(a.shape, lambda i: (0, 0))
    ws = [wn1h, wn1a, bn1.reshape(1, -1), wn2, bn2.reshape(1, -1),
          wn3, bn3.reshape(1, -1)]
    return pl.pallas_call(
        functools.partial(_node_body, residual),
        grid=(_N // block_rows,),
        in_specs=[
            pl.BlockSpec((block_rows, _H), lambda i: (i, 0)),
            pl.BlockSpec((block_rows, _H), lambda i: (i, 0)),
            pl.BlockSpec((block_rows, _H), lambda i: (i, 0)),
        ] + [full(w) for w in ws],
        out_specs=pl.BlockSpec((block_rows, _H), lambda i: (i, 0)),
        out_shape=jax.ShapeDtypeStruct((_N, _H), F32),
    )(h, p0, p1, *ws)


def _dec_body(h_ref, w1, b1, w2, b2, w3, b3,
              g0, g1, be0, be1, gw1a, gw1b, gb1, gw2, gb2, gw3, gb3,
              np_ref, gp_ref):
    h = h_ref[...]
    a = _relu((_dot(h, w1[...]) + b1[...]) * _BN)
    a = _relu((_dot(a, w2[...]) + b2[...]) * _BN)
    np_ref[...] = _dot(a, w3[...]) + b3[...]

    xm = jnp.mean(h, axis=0, keepdims=True)
    xx = jnp.max(h, axis=0, keepdims=True)
    mu = (jnp.sum(xm) + jnp.sum(xx)) / 256.0
    var = (jnp.sum((xm - mu) ** 2) + jnp.sum((xx - mu) ** 2)) / 256.0
    inv = lax.rsqrt(var + 1e-5)
    gf0 = (xm - mu) * inv * g0[...] + be0[...]
    gf1 = (xx - mu) * inv * g1[...] + be1[...]
    g = _relu((_dot(gf0, gw1a[...]) + _dot(gf1, gw1b[...]) + gb1[...]) * _BN)
    g = _relu((_dot(g, gw2[...]) + gb2[...]) * _BN)
    gp_ref[...] = _dot(g, gw3[...]) + gb3[...]


def _tc_decoder(h, node_dec, ln_gamma, ln_beta, glob_dec):
    (w1, b1), (w2, b2), (w3, b3) = node_dec
    (gw1, gb1), (gw2, gb2), (gw3, gb3) = glob_dec
    ws = [w1, b1.reshape(1, -1), w2, b2.reshape(1, -1), w3, b3.reshape(1, -1),
          ln_gamma[:_H].reshape(1, -1), ln_gamma[_H:].reshape(1, -1),
          ln_beta[:_H].reshape(1, -1), ln_beta[_H:].reshape(1, -1),
          gw1[:_H], gw1[_H:], gb1.reshape(1, -1), gw2, gb2.reshape(1, -1),
          gw3, gb3.reshape(1, -1)]
    return pl.pallas_call(
        _dec_body,
        out_shape=[
            jax.ShapeDtypeStruct((_N, 3), F32),
            jax.ShapeDtypeStruct((1, 2), F32),
        ],
    )(h, *ws)


def _sc_gather(h, row_r, col_r):
    @functools.partial(
        pl.kernel,
        out_type=(jax.ShapeDtypeStruct((_E, _H), F32),
                  jax.ShapeDtypeStruct((_E, _H), F32)),
        mesh=_vmesh,
        scratch_types=[
            pltpu.VMEM((_NCH, _C), jnp.int32),
            pltpu.VMEM((_NCH, _C), jnp.int32),
            pltpu.VMEM((_C, _H), F32),
            pltpu.VMEM((_C, _H), F32),
            pltpu.SemaphoreType.DMA,
            pltpu.SemaphoreType.DMA,
        ],
    )
    def k(h_hbm, row_hbm, col_hbm, hr_hbm, hc_hbm,
          ridx, cidx, bufr, bufc, sem1, sem2):
        wid = lax.axis_index("s") * 2 + lax.axis_index("c")
        base = wid * _EPW
        pltpu.sync_copy(row_hbm.at[wid], ridx)
        pltpu.sync_copy(col_hbm.at[wid], cidx)

        @pl.loop(0, _NCH)
        def _(ch):
            cp_r = pltpu.async_copy(h_hbm.at[ridx.at[ch]], bufr, sem1)
            cp_c = pltpu.async_copy(h_hbm.at[cidx.at[ch]], bufc, sem2)
            cp_r.wait()
            cp_c.wait()
            off = base + ch * _C
            pltpu.sync_copy(bufr, hr_hbm.at[pl.ds(off, _C)])
            pltpu.sync_copy(bufc, hc_hbm.at[pl.ds(off, _C)])

    return k(h, row_r, col_r)


def _sc_scatter(msgs, col_r):
    @functools.partial(
        pl.kernel,
        out_type=jax.ShapeDtypeStruct((2, _NPAD, _H), F32),
        mesh=_vmesh,
        scratch_types=[
            pltpu.VMEM((_NCH, _C), jnp.int32),
            pltpu.VMEM((_C, _H), F32),
            pltpu.VMEM((_ZR, _H), F32),
            pltpu.VMEM_SHARED((_NPAD, _H), F32),
            pltpu.SemaphoreType.DMA,
        ],
    )
    def k(m_hbm, col_hbm, out_hbm, cidx, buf, zbuf, acc, sem):
        cid = lax.axis_index("c")
        sid = lax.axis_index("s")
        wid = sid * 2 + cid
        base = wid * _EPW
        pltpu.sync_copy(col_hbm.at[wid], cidx)

        @pl.loop(0, _ZR)
        def _(r):
            @pl.loop(0, _H // 16)
            def _(c16):
                zbuf[r, pl.ds(c16 * 16, 16)] = jnp.zeros((16,), F32)

        @pl.loop(0, _NPW // _ZR)
        def _(j):
            pltpu.sync_copy(zbuf, acc.at[pl.ds(sid * _NPW + j * _ZR, _ZR)])

        plsc.subcore_barrier()

        @pl.loop(0, _NCH)
        def _(ch):
            pltpu.sync_copy(m_hbm.at[pl.ds(base + ch * _C, _C)], buf)
            pltpu.sync_copy(buf, acc.at[cidx.at[ch]], add=True)

        plsc.subcore_barrier()

        @pl.loop(0, _NPW // _ZR)
        def _(j):
            off = sid * _NPW + j * _ZR
            pltpu.sync_copy(acc.at[pl.ds(off, _ZR)], zbuf)
            pltpu.sync_copy(zbuf, out_hbm.at[cid, pl.ds(off, _ZR)])

    return k(msgs, col_r)


def kernel(x, edge_index, edge_attr, params):
    row = edge_index[0]
    col = edge_index[1]
    row_r = row.reshape(_NW, _NCH, _C)
    col_r = col.reshape(_NW, _NCH, _C)

    h = _mlp3(x, params['node_encoder'], block_rows=2000)
    e = _mlp3(edge_attr, params['edge_encoder'], block_rows=8000)

    for i, lp in enumerate(params['layers']):
        hr, hc = _sc_gather(h, row_r, col_r)
        e_upd, msgs = _tc_edge(hr, hc, e, lp['edge_mlp'], lp['edge_update_mlp'])
        partials = _sc_scatter(msgs, col_r)
        h = _tc_node(h, partials[0, :_N], partials[1, :_N], lp['node_mlp'],
                     residual=(i > 0))
        e = e_upd

    node_pred, global_pred = _tc_decoder(
        h, params['node_decoder'], params['ln_gamma'], params['ln_beta'],
        params['global_decoder'])
    return (node_pred, global_pred)


# speedup vs baseline: 3.0786x; 1.0896x over previous
import functools

import jax
import jax.numpy as jnp
import numpy as np
from jax import lax
from jax.experimental import pallas as pl
from jax.experimental.pallas import tpu as pltpu
from jax.experimental.pallas import tpu_sc as plsc

F32 = jnp.float32
_N = 10000
_E = 320000
_H = 128
_BN = np.float32(1.0 / np.sqrt(1.0 + 1e-5))

_S = 2
_ES = _E // _S
_NW = 32
_EPT = _ES // _NW
_C = 40
_NCH = _EPT // _C
_NPAD = 10240
_NPW = _NPAD // 16
_ZR = 128

_vmesh = plsc.VectorSubcoreMesh(core_axis_name="c", subcore_axis_name="s")


def _dot(a, b):
    return jnp.dot(a, b, preferred_element_type=F32)


def _relu(x):
    return jnp.maximum(x, 0.0)


def _mlp3_body(x_ref, w1, b1, w2, b2, w3, b3, o_ref):
    a = _relu((_dot(x_ref[...], w1[...]) + b1[...]) * _BN)
    a = _relu((_dot(a, w2[...]) + b2[...]) * _BN)
    o_ref[...] = _dot(a, w3[...]) + b3[...]


def _mlp3(xarr, mlp, block_rows):
    (w1, b1), (w2, b2), (w3, b3) = mlp
    rows = xarr.shape[0]
    dout = w3.shape[1]
    ws = [w1, b1.reshape(1, -1), w2, b2.reshape(1, -1), w3, b3.reshape(1, -1)]
    full = lambda a: pl.BlockSpec(a.shape, lambda i: (0, 0))
    return pl.pallas_call(
        _mlp3_body,
        grid=(rows // block_rows,),
        in_specs=[
            pl.BlockSpec((block_rows, xarr.shape[1]), lambda i: (i, 0)),
        ] + [full(w) for w in ws],
        out_specs=pl.BlockSpec((block_rows, dout), lambda i: (i, 0)),
        out_shape=jax.ShapeDtypeStruct((rows, dout), F32),
    )(xarr, *ws)


def _edge_body(hr_ref, hc_ref, e_ref,
               w1a, w1b, w1c, b1, w2, b2, w3, b3,
               wu1e, wu1m, bu1, wu2, bu2, wu3, bu3,
               eupd_ref, msgs_ref):
    hr = hr_ref[...]
    hc = hc_ref[...]
    e = e_ref[...]

    t1 = _dot(hr, w1a[...]) + _dot(hc, w1b[...]) + _dot(e, w1c[...]) + b1[...]
    a = _relu(t1 * _BN)
    a = _relu((_dot(a, w2[...]) + b2[...]) * _BN)
    em = _dot(a, w3[...]) + b3[...]

    u = _relu((_dot(e, wu1e[...]) + _dot(em, wu1m[...]) + bu1[...]) * _BN)
    u = _relu((_dot(u, wu2[...]) + bu2[...]) * _BN)
    e_upd = _dot(u, wu3[...]) + bu3[...]
    eupd_ref[...] = e_upd

    t2 = _dot(hc, w1a[...]) + _dot(hr, w1b[...]) + _dot(e_upd, w1c[...]) + b1[...]
    m = _relu(t2 * _BN)
    m = _relu((_dot(m, w2[...]) + b2[...]) * _BN)
    msgs_ref[...] = _dot(m, w3[...]) + b3[...]


def _tc_edge(hr, hc, e, edge_mlp, edge_update_mlp, block_rows=4000):
    (w1, b1), (w2, b2), (w3, b3) = edge_mlp
    (wu1, bu1), (wu2, bu2), (wu3, bu3) = edge_update_mlp
    w1a, w1b, w1c = w1[:_H], w1[_H:2 * _H], w1[2 * _H:]
    wu1e, wu1m = wu1[:_H // 2], wu1[_H // 2:]
    full = lambda a: pl.BlockSpec(a.shape, lambda i: (0, 0))
    ws = [w1a, w1b, w1c, b1.reshape(1, -1), w2, b2.reshape(1, -1),
          w3, b3.reshape(1, -1),
          wu1e, wu1m, bu1.reshape(1, -1), wu2, bu2.reshape(1, -1),
          wu3, bu3.reshape(1, -1)]
    rows = hr.shape[0]
    return pl.pallas_call(
        _edge_body,
        grid=(rows // block_rows,),
        in_specs=[
            pl.BlockSpec((block_rows, _H), lambda i: (i, 0)),
            pl.BlockSpec((block_rows, _H), lambda i: (i, 0)),
            pl.BlockSpec((block_rows, _H // 2), lambda i: (i, 0)),
        ] + [full(w) for w in ws],
        out_specs=[
            pl.BlockSpec((block_rows, _H // 2), lambda i: (i, 0)),
            pl.BlockSpec((block_rows, _H), lambda i: (i, 0)),
        ],
        out_shape=[
            jax.ShapeDtypeStruct((rows, _H // 2), F32),
            jax.ShapeDtypeStruct((rows, _H), F32),
        ],
    )(hr, hc, e, *ws)


def _node_body(res, h_ref, p0_ref, p1_ref, p2_ref, p3_ref,
               wn1h, wn1a, bn1, wn2, bn2, wn3, bn3, o_ref):
    h = h_ref[...]
    agg = (p0_ref[...] + p1_ref[...]) + (p2_ref[...] + p3_ref[...])
    a = _relu((_dot(h, wn1h[...]) + _dot(agg, wn1a[...]) + bn1[...]) * _BN)
    a = _relu((_dot(a, wn2[...]) + bn2[...]) * _BN)
    out = _dot(a, wn3[...]) + bn3[...]
    if res:
        out = out + h
    o_ref[...] = out


def _tc_node(h, parts, node_mlp, residual, block_rows=2000):
    (wn1, bn1), (wn2, bn2), (wn3, bn3) = node_mlp
    wn1h, wn1a = wn1[:_H], wn1[_H:]
    full = lambda a: pl.BlockSpec(a.shape, lambda i: (0, 0))
    ws = [wn1h, wn1a, bn1.reshape(1, -1), wn2, bn2.reshape(1, -1),
          wn3, bn3.reshape(1, -1)]
    return pl.pallas_call(
        functools.partial(_node_body, residual),
        grid=(_N // block_rows,),
        in_specs=[pl.BlockSpec((block_rows, _H), lambda i: (i, 0))] * 5
        + [full(w) for w in ws],
        out_specs=pl.BlockSpec((block_rows, _H), lambda i: (i, 0)),
        out_shape=jax.ShapeDtypeStruct((_N, _H), F32),
    )(h, *parts, *ws)


def _dec_body(h_ref, w1, b1, w2, b2, w3, b3,
              g0, g1, be0, be1, gw1a, gw1b, gb1, gw2, gb2, gw3, gb3,
              np_ref, gp_ref):
    h = h_ref[...]
    a = _relu((_dot(h, w1[...]) + b1[...]) * _BN)
    a = _relu((_dot(a, w2[...]) + b2[...]) * _BN)
    np_ref[...] = _dot(a, w3[...]) + b3[...]

    xm = jnp.mean(h, axis=0, keepdims=True)
    xx = jnp.max(h, axis=0, keepdims=True)
    mu = (jnp.sum(xm) + jnp.sum(xx)) / 256.0
    var = (jnp.sum((xm - mu) ** 2) + jnp.sum((xx - mu) ** 2)) / 256.0
    inv = lax.rsqrt(var + 1e-5)
    gf0 = (xm - mu) * inv * g0[...] + be0[...]
    gf1 = (xx - mu) * inv * g1[...] + be1[...]
    g = _relu((_dot(gf0, gw1a[...]) + _dot(gf1, gw1b[...]) + gb1[...]) * _BN)
    g = _relu((_dot(g, gw2[...]) + gb2[...]) * _BN)
    gp_ref[...] = _dot(g, gw3[...]) + gb3[...]


def _tc_decoder(h, node_dec, ln_gamma, ln_beta, glob_dec):
    (w1, b1), (w2, b2), (w3, b3) = node_dec
    (gw1, gb1), (gw2, gb2), (gw3, gb3) = glob_dec
    ws = [w1, b1.reshape(1, -1), w2, b2.reshape(1, -1), w3, b3.reshape(1, -1),
          ln_gamma[:_H].reshape(1, -1), ln_gamma[_H:].reshape(1, -1),
          ln_beta[:_H].reshape(1, -1), ln_beta[_H:].reshape(1, -1),
          gw1[:_H], gw1[_H:], gb1.reshape(1, -1), gw2, gb2.reshape(1, -1),
          gw3, gb3.reshape(1, -1)]
    return pl.pallas_call(
        _dec_body,
        out_shape=[
            jax.ShapeDtypeStruct((_N, 3), F32),
            jax.ShapeDtypeStruct((1, 2), F32),
        ],
    )(h, *ws)


def _sc_gather(h, row_r, col_r):
    @functools.partial(
        pl.kernel,
        out_type=(jax.ShapeDtypeStruct((_ES, _H), F32),
                  jax.ShapeDtypeStruct((_ES, _H), F32)),
        mesh=_vmesh,
        scratch_types=[
            pltpu.VMEM((_NCH, _C), jnp.int32),
            pltpu.VMEM((_NCH, _C), jnp.int32),
            pltpu.VMEM((_C, _H), F32),
            pltpu.VMEM((_C, _H), F32),
            pltpu.SemaphoreType.DMA,
            pltpu.SemaphoreType.DMA,
        ],
    )
    def k(h_hbm, row_hbm, col_hbm, hr_hbm, hc_hbm,
          ridx, cidx, bufr, bufc, sem1, sem2):
        wid = lax.axis_index("s") * 2 + lax.axis_index("c")
        base = wid * _EPT
        pltpu.sync_copy(row_hbm.at[wid], ridx)
        pltpu.sync_copy(col_hbm.at[wid], cidx)

        @pl.loop(0, _NCH)
        def _(ch):
            cp_r = pltpu.async_copy(h_hbm.at[ridx.at[ch]], bufr, sem1)
            cp_c = pltpu.async_copy(h_hbm.at[cidx.at[ch]], bufc, sem2)
            cp_r.wait()
            cp_c.wait()
            off = base + ch * _C
            pltpu.sync_copy(bufr, hr_hbm.at[pl.ds(off, _C)])
            pltpu.sync_copy(bufc, hc_hbm.at[pl.ds(off, _C)])

    return k(h, row_r, col_r)


def _sc_scatter(msgs, col_r):
    @functools.partial(
        pl.kernel,
        out_type=jax.ShapeDtypeStruct((2, _NPAD, _H), F32),
        mesh=_vmesh,
        scratch_types=[
            pltpu.VMEM((_NCH, _C), jnp.int32),
            pltpu.VMEM((_C, _H), F32),
            pltpu.VMEM((_ZR, _H), F32),
            pltpu.VMEM_SHARED((_NPAD, _H), F32),
            pltpu.SemaphoreType.DMA,
        ],
    )
    def k(m_hbm, col_hbm, out_hbm, cidx, buf, zbuf, acc, sem):
        cid = lax.axis_index("c")
        sid = lax.axis_index("s")
        wid = sid * 2 + cid
        base = wid * _EPT
        pltpu.sync_copy(col_hbm.at[wid], cidx)

        @pl.loop(0, _ZR)
        def _(r):
            @pl.loop(0, _H // 16)
            def _(c16):
                zbuf[r, pl.ds(c16 * 16, 16)] = jnp.zeros((16,), F32)

        @pl.loop(0, _NPW // _ZR)
        def _(j):
            pltpu.sync_copy(zbuf, acc.at[pl.ds(sid * _NPW + j * _ZR, _ZR)])

        plsc.subcore_barrier()

        @pl.loop(0, _NCH)
        def _(ch):
            pltpu.sync_copy(m_hbm.at[pl.ds(base + ch * _C, _C)], buf)
            pltpu.sync_copy(buf, acc.at[cidx.at[ch]], add=True)

        plsc.subcore_barrier()

        @pl.loop(0, _NPW // _ZR)
        def _(j):
            off = sid * _NPW + j * _ZR
            pltpu.sync_copy(acc.at[pl.ds(off, _ZR)], zbuf)
            pltpu.sync_copy(zbuf, out_hbm.at[cid, pl.ds(off, _ZR)])

    return k(msgs, col_r)


def kernel(x, edge_index, edge_attr, params):
    row_r = edge_index[0].reshape(_S, _NW, _NCH, _C)
    col_r = edge_index[1].reshape(_S, _NW, _NCH, _C)

    h = _mlp3(x, params['node_encoder'], block_rows=2000)
    e_parts = [_mlp3(edge_attr[s * _ES:(s + 1) * _ES],
                     params['edge_encoder'], block_rows=8000)
               for s in range(_S)]

    for i, lp in enumerate(params['layers']):
        gathered = [_sc_gather(h, row_r[s], col_r[s]) for s in range(_S)]
        em = [_tc_edge(gathered[s][0], gathered[s][1], e_parts[s],
                       lp['edge_mlp'], lp['edge_update_mlp'])
              for s in range(_S)]
        parts = [_sc_scatter(em[s][1], col_r[s]) for s in range(_S)]
        h = _tc_node(h,
                     [parts[0][0, :_N], parts[0][1, :_N],
                      parts[1][0, :_N], parts[1][1, :_N]],
                     lp['node_mlp'], residual=(i > 0))
        e_parts = [em[s][0] for s in range(_S)]

    node_pred, global_pred = _tc_decoder(
        h, params['node_decoder'], params['ln_gamma'], params['ln_beta'],
        params['global_decoder'])
    return (node_pred, global_pred)


# speedup vs baseline: 3.5453x; 1.1516x over previous
import functools

import jax
import jax.numpy as jnp
import numpy as np
from jax import lax
from jax.experimental import pallas as pl
from jax.experimental.pallas import tpu as pltpu
from jax.experimental.pallas import tpu_sc as plsc

F32 = jnp.float32
_N = 10000
_E = 320000
_H = 128
_BN = np.float32(1.0 / np.sqrt(1.0 + 1e-5))

_S = 2
_NW = 32
_C = 80
_NCHS = (64, 61)
_EPTS = (_NCHS[0] * _C, _NCHS[1] * _C)
_ROWS = (_EPTS[0] * _NW, _EPTS[1] * _NW)
_NPAD = 10240
_NPW = _NPAD // 16
_ZR = 128

_vmesh = plsc.VectorSubcoreMesh(core_axis_name="c", subcore_axis_name="s")


def _dot(a, b):
    return jnp.dot(a, b, preferred_element_type=F32)


def _relu(x):
    return jnp.maximum(x, 0.0)


def _mlp3_body(x_ref, w1, b1, w2, b2, w3, b3, o_ref):
    a = _relu((_dot(x_ref[...], w1[...]) + b1[...]) * _BN)
    a = _relu((_dot(a, w2[...]) + b2[...]) * _BN)
    o_ref[...] = _dot(a, w3[...]) + b3[...]


def _mlp3(xarr, mlp, block_rows):
    (w1, b1), (w2, b2), (w3, b3) = mlp
    rows = xarr.shape[0]
    dout = w3.shape[1]
    ws = [w1, b1.reshape(1, -1), w2, b2.reshape(1, -1), w3, b3.reshape(1, -1)]
    full = lambda a: pl.BlockSpec(a.shape, lambda i: (0, 0))
    return pl.pallas_call(
        _mlp3_body,
        grid=(rows // block_rows,),
        in_specs=[
            pl.BlockSpec((block_rows, xarr.shape[1]), lambda i: (i, 0)),
        ] + [full(w) for w in ws],
        out_specs=pl.BlockSpec((block_rows, dout), lambda i: (i, 0)),
        out_shape=jax.ShapeDtypeStruct((rows, dout), F32),
    )(xarr, *ws)


def _edge_body(hr_ref, hc_ref, e_ref,
               w1a, w1b, w1c, b1, w2, b2, w3, b3,
               wu1e, wu1m, bu1, wu2, bu2, wu3, bu3,
               eupd_ref, msgs_ref):
    hr = hr_ref[...]
    hc = hc_ref[...]
    e = e_ref[...]

    t1 = _dot(hr, w1a[...]) + _dot(hc, w1b[...]) + _dot(e, w1c[...]) + b1[...]
    a = _relu(t1 * _BN)
    a = _relu((_dot(a, w2[...]) + b2[...]) * _BN)
    em = _dot(a, w3[...]) + b3[...]

    u = _relu((_dot(e, wu1e[...]) + _dot(em, wu1m[...]) + bu1[...]) * _BN)
    u = _relu((_dot(u, wu2[...]) + bu2[...]) * _BN)
    e_upd = _dot(u, wu3[...]) + bu3[...]
    eupd_ref[...] = e_upd

    t2 = _dot(hc, w1a[...]) + _dot(hr, w1b[...]) + _dot(e_upd, w1c[...]) + b1[...]
    m = _relu(t2 * _BN)
    m = _relu((_dot(m, w2[...]) + b2[...]) * _BN)
    msgs_ref[...] = _dot(m, w3[...]) + b3[...]


def _tc_edge(hr, hc, e, edge_mlp, edge_update_mlp, block_rows=4000):
    (w1, b1), (w2, b2), (w3, b3) = edge_mlp
    (wu1, bu1), (wu2, bu2), (wu3, bu3) = edge_update_mlp
    w1a, w1b, w1c = w1[:_H], w1[_H:2 * _H], w1[2 * _H:]
    wu1e, wu1m = wu1[:_H // 2], wu1[_H // 2:]
    full = lambda a: pl.BlockSpec(a.shape, lambda i: (0, 0))
    ws = [w1a, w1b, w1c, b1.reshape(1, -1), w2, b2.reshape(1, -1),
          w3, b3.reshape(1, -1),
          wu1e, wu1m, bu1.reshape(1, -1), wu2, bu2.reshape(1, -1),
          wu3, bu3.reshape(1, -1)]
    rows = hr.shape[0]
    return pl.pallas_call(
        _edge_body,
        grid=(rows // block_rows,),
        in_specs=[
            pl.BlockSpec((block_rows, _H), lambda i: (i, 0)),
            pl.BlockSpec((block_rows, _H), lambda i: (i, 0)),
            pl.BlockSpec((block_rows, _H // 2), lambda i: (i, 0)),
        ] + [full(w) for w in ws],
        out_specs=[
            pl.BlockSpec((block_rows, _H // 2), lambda i: (i, 0)),
            pl.BlockSpec((block_rows, _H), lambda i: (i, 0)),
        ],
        out_shape=[
            jax.ShapeDtypeStruct((rows, _H // 2), F32),
            jax.ShapeDtypeStruct((rows, _H), F32),
        ],
    )(hr, hc, e, *ws)


def _node_body(res, h_ref, p0_ref, p1_ref, p2_ref, p3_ref,
               wn1h, wn1a, bn1, wn2, bn2, wn3, bn3, o_ref):
    h = h_ref[...]
    agg = (p0_ref[...] + p1_ref[...]) + (p2_ref[...] + p3_ref[...])
    a = _relu((_dot(h, wn1h[...]) + _dot(agg, wn1a[...]) + bn1[...]) * _BN)
    a = _relu((_dot(a, wn2[...]) + bn2[...]) * _BN)
    out = _dot(a, wn3[...]) + bn3[...]
    if res:
        out = out + h
    o_ref[...] = out


def _tc_node(h, parts, node_mlp, residual, block_rows=2000):
    (wn1, bn1), (wn2, bn2), (wn3, bn3) = node_mlp
    wn1h, wn1a = wn1[:_H], wn1[_H:]
    full = lambda a: pl.BlockSpec(a.shape, lambda i: (0, 0))
    ws = [wn1h, wn1a, bn1.reshape(1, -1), wn2, bn2.reshape(1, -1),
          wn3, bn3.reshape(1, -1)]
    return pl.pallas_call(
        functools.partial(_node_body, residual),
        grid=(_N // block_rows,),
        in_specs=[pl.BlockSpec((block_rows, _H), lambda i: (i, 0))] * 5
        + [full(w) for w in ws],
        out_specs=pl.BlockSpec((block_rows, _H), lambda i: (i, 0)),
        out_shape=jax.ShapeDtypeStruct((_N, _H), F32),
    )(h, *parts, *ws)


def _dec_body(h_ref, w1, b1, w2, b2, w3, b3,
              g0, g1, be0, be1, gw1a, gw1b, gb1, gw2, gb2, gw3, gb3,
              np_ref, gp_ref):
    h = h_ref[...]
    a = _relu((_dot(h, w1[...]) + b1[...]) * _BN)
    a = _relu((_dot(a, w2[...]) + b2[...]) * _BN)
    np_ref[...] = _dot(a, w3[...]) + b3[...]

    xm = jnp.mean(h, axis=0, keepdims=True)
    xx = jnp.max(h, axis=0, keepdims=True)
    mu = (jnp.sum(xm) + jnp.sum(xx)) / 256.0
    var = (jnp.sum((xm - mu) ** 2) + jnp.sum((xx - mu) ** 2)) / 256.0
    inv = lax.rsqrt(var + 1e-5)
    gf0 = (xm - mu) * inv * g0[...] + be0[...]
    gf1 = (xx - mu) * inv * g1[...] + be1[...]
    g = _relu((_dot(gf0, gw1a[...]) + _dot(gf1, gw1b[...]) + gb1[...]) * _BN)
    g = _relu((_dot(g, gw2[...]) + gb2[...]) * _BN)
    gp_ref[...] = _dot(g, gw3[...]) + gb3[...]


def _tc_decoder(h, node_dec, ln_gamma, ln_beta, glob_dec):
    (w1, b1), (w2, b2), (w3, b3) = node_dec
    (gw1, gb1), (gw2, gb2), (gw3, gb3) = glob_dec
    ws = [w1, b1.reshape(1, -1), w2, b2.reshape(1, -1), w3, b3.reshape(1, -1),
          ln_gamma[:_H].reshape(1, -1), ln_gamma[_H:].reshape(1, -1),
          ln_beta[:_H].reshape(1, -1), ln_beta[_H:].reshape(1, -1),
          gw1[:_H], gw1[_H:], gb1.reshape(1, -1), gw2, gb2.reshape(1, -1),
          gw3, gb3.reshape(1, -1)]
    return pl.pallas_call(
        _dec_body,
        out_shape=[
            jax.ShapeDtypeStruct((_N, 3), F32),
            jax.ShapeDtypeStruct((1, 2), F32),
        ],
    )(h, *ws)


def _sc_gather(h, row_r, col_r, nch):
    ept = nch * _C
    rows = _NW * ept

    @functools.partial(
        pl.kernel,
        out_type=(jax.ShapeDtypeStruct((rows, _H), F32),
                  jax.ShapeDtypeStruct((rows, _H), F32)),
        mesh=_vmesh,
        scratch_types=[
            pltpu.VMEM((nch, _C), jnp.int32),
            pltpu.VMEM((nch, _C), jnp.int32),
            pltpu.VMEM((_C, _H), F32),
            pltpu.VMEM((_C, _H), F32),
            pltpu.VMEM((_C, _H), F32),
            pltpu.VMEM((_C, _H), F32),
            pltpu.SemaphoreType.DMA,
            pltpu.SemaphoreType.DMA,
            pltpu.SemaphoreType.DMA,
            pltpu.SemaphoreType.DMA,
        ],
    )
    def k(h_hbm, row_hbm, col_hbm, hr_hbm, hc_hbm,
          ridx, cidx, br0, bc0, br1, bc1, sg0, sg1, sw0, sw1):
        wid = lax.axis_index("s") * 2 + lax.axis_index("c")
        base = wid * ept
        pltpu.sync_copy(row_hbm.at[wid], ridx)
        pltpu.sync_copy(col_hbm.at[wid], cidx)
        bufr = (br0, br1)
        bufc = (bc0, bc1)
        sg = (sg0, sg1)
        sw = (sw0, sw1)

        def wait_gather(b):
            pltpu.make_async_copy(h_hbm.at[pl.ds(0, _C)], bufr[b], sg[b]).wait()
            pltpu.make_async_copy(h_hbm.at[pl.ds(0, _C)], bufc[b], sg[b]).wait()

        def wait_writeback(b):
            pltpu.make_async_copy(bufr[b], hr_hbm.at[pl.ds(0, _C)], sw[b]).wait()
            pltpu.make_async_copy(bufc[b], hc_hbm.at[pl.ds(0, _C)], sw[b]).wait()

        def start_gather(ch, b):
            pltpu.async_copy(h_hbm.at[ridx.at[ch]], bufr[b], sg[b])
            pltpu.async_copy(h_hbm.at[cidx.at[ch]], bufc[b], sg[b])

        def slot(ch, b):
            wait_gather(b)
            bb = 1 - b

            @pl.when(ch + 1 < nch)
            def _():
                @pl.when(ch >= 1)
                def _():
                    wait_writeback(bb)
                start_gather(ch + 1, bb)

            off = base + ch * _C
            pltpu.async_copy(bufr[b], hr_hbm.at[pl.ds(off, _C)], sw[b])
            pltpu.async_copy(bufc[b], hc_hbm.at[pl.ds(off, _C)], sw[b])

        start_gather(0, 0)

        @pl.loop(0, (nch + 1) // 2)
        def _(i):
            for b in (0, 1):
                ch = i * 2 + b

                @pl.when(ch < nch)
                def _(ch=ch, b=b):
                    slot(ch, b)

        wait_writeback(0)
        wait_writeback(1)

    return k(h, row_r, col_r)


def _sc_scatter(msgs, col_r, nch):
    ept = nch * _C

    @functools.partial(
        pl.kernel,
        out_type=jax.ShapeDtypeStruct((2, _NPAD, _H), F32),
        mesh=_vmesh,
        scratch_types=[
            pltpu.VMEM((nch, _C), jnp.int32),
            pltpu.VMEM((_C, _H), F32),
            pltpu.VMEM((_C, _H), F32),
            pltpu.VMEM((_ZR, _H), F32),
            pltpu.VMEM_SHARED((_NPAD, _H), F32),
            pltpu.SemaphoreType.DMA,
            pltpu.SemaphoreType.DMA,
        ],
    )
    def k(m_hbm, col_hbm, out_hbm, cidx, b0, b1, zbuf, acc, sl0, sl1):
        cid = lax.axis_index("c")
        sid = lax.axis_index("s")
        wid = sid * 2 + cid
        base = wid * ept
        buf = (b0, b1)
        sl = (sl0, sl1)
        pltpu.sync_copy(col_hbm.at[wid], cidx)

        @pl.loop(0, _ZR)
        def _(r):
            @pl.loop(0, _H // 16)
            def _(c16):
                zbuf[r, pl.ds(c16 * 16, 16)] = jnp.zeros((16,), F32)

        @pl.loop(0, _NPW // _ZR)
        def _(j):
            pltpu.sync_copy(zbuf, acc.at[pl.ds(sid * _NPW + j * _ZR, _ZR)])

        plsc.subcore_barrier()

        pltpu.async_copy(m_hbm.at[pl.ds(base, _C)], b0, sl0)

        def slot(ch, b):
            pltpu.make_async_copy(m_hbm.at[pl.ds(0, _C)], buf[b], sl[b]).wait()
            bb = 1 - b

            @pl.when(ch + 1 < nch)
            def _():
                pltpu.async_copy(
                    m_hbm.at[pl.ds(base + (ch + 1) * _C, _C)], buf[bb], sl[bb])

            pltpu.sync_copy(buf[b], acc.at[cidx.at[ch]], add=True)

        @pl.loop(0, (nch + 1) // 2)
        def _(i):
            for b in (0, 1):
                ch = i * 2 + b

                @pl.when(ch < nch)
                def _(ch=ch, b=b):
                    slot(ch, b)

        plsc.subcore_barrier()

        @pl.loop(0, _NPW // _ZR)
        def _(j):
            off = sid * _NPW + j * _ZR
            pltpu.sync_copy(acc.at[pl.ds(off, _ZR)], zbuf)
            pltpu.sync_copy(zbuf, out_hbm.at[cid, pl.ds(off, _ZR)])

    return k(msgs, col_r)


def kernel(x, edge_index, edge_attr, params):
    row = edge_index[0]
    col = edge_index[1]
    bounds = (0, _ROWS[0], _E)
    row_r = [row[bounds[s]:bounds[s + 1]].reshape(_NW, _NCHS[s], _C)
             for s in range(_S)]
    col_r = [col[bounds[s]:bounds[s + 1]].reshape(_NW, _NCHS[s], _C)
             for s in range(_S)]

    h = _mlp3(x, params['node_encoder'], block_rows=2000)
    enc_blocks = (8192, 7808)
    e_parts = [_mlp3(edge_attr[bounds[s]:bounds[s + 1]],
                     params['edge_encoder'], block_rows=enc_blocks[s])
               for s in range(_S)]

    edge_blocks = (4096, 4880)
    for i, lp in enumerate(params['layers']):
        gathered = [_sc_gather(h, row_r[s], col_r[s], _NCHS[s])
                    for s in range(_S)]
        em = [_tc_edge(gathered[s][0], gathered[s][1], e_parts[s],
                       lp['edge_mlp'], lp['edge_update_mlp'],
                       block_rows=edge_blocks[s])
              for s in range(_S)]
        parts = [_sc_scatter(em[s][1], col_r[s], _NCHS[s]) for s in range(_S)]
        h = _tc_node(h,
                     [parts[0][0, :_N], parts[0][1, :_N],
                      parts[1][0, :_N], parts[1][1, :_N]],
                     lp['node_mlp'], residual=(i > 0))
        e_parts = [em[s][0] for s in range(_S)]

    node_pred, global_pred = _tc_decoder(
        h, params['node_decoder'], params['ln_gamma'], params['ln_beta'],
        params['global_decoder'])
    return (node_pred, global_pred)
